# Initial kernel scaffold; baseline (speedup 1.0000x reference)
import jax
import jax.numpy as jnp
from jax.experimental import pallas as pl


def kernel(x, edge_index, node_coords, batch, W1l, b1l, W1r, W2l, b2l, W2r, ln1g, ln1b, ln2g, ln2b, Bffm, fcW, fcb, flng, flnb):
    raise NotImplementedError("write your pallas kernel here")


# speedup vs baseline: 4.6346x; 4.6346x over previous
import functools

import jax
import jax.numpy as jnp
from jax import lax
from jax.experimental import pallas as pl
from jax.experimental.pallas import tpu as pltpu
from jax.experimental.pallas import tpu_sc as plsc

_N = 50000
_E = 800000
_G = 64
_BN = 256
_NB = 196
_NTC = _NB * _BN
_NPAD = 51200
_ZR = _NPAD // 16
_ER = 6400
_EPAD = _ER * 128
_K = 4
_SCALES = (1.0, 5.0, 10.0, 30.0)

def _sc_mesh():
  return plsc.VectorSubcoreMesh(
      core_axis_name="c", subcore_axis_name="s", num_cores=2, num_subcores=16)


def _edge_loop(tbl, srcR, dstR, acc, sbuf, dbuf, rows, gsem, ssem,
               base_row, n_outer):
  def outer(i, carry):
    row0 = base_row + i * _K
    pltpu.sync_copy(srcR.at[pl.ds(row0, _K)], sbuf)
    pltpu.sync_copy(dstR.at[pl.ds(row0, _K)], dbuf)
    gd = [pltpu.async_copy(tbl.at[sbuf.at[j]], rows.at[j], gsem)
          for j in range(_K)]
    for d in gd:
      d.wait()
    sd = [pltpu.async_copy(rows.at[j], acc.at[dbuf.at[j]], ssem, add=True)
          for j in range(_K)]
    for d in sd:
      d.wait()
    return carry

  lax.fori_loop(0, n_outer, outer, 0)


def _sc_layer1(xp, srcR, dstR, z8):
  @functools.partial(
      pl.kernel,
      out_type=jax.ShapeDtypeStruct((2, _NPAD, 8), jnp.float32),
      mesh=_sc_mesh(),
      compiler_params=pltpu.CompilerParams(use_tc_tiling_on_sc=False),
      scratch_types=[
          pltpu.VMEM_SHARED((_NPAD, 8), jnp.float32),
          pltpu.VMEM((_K, 128), jnp.int32),
          pltpu.VMEM((_K, 128), jnp.int32),
          pltpu.VMEM((_K, 128, 8), jnp.float32),
          pltpu.SemaphoreType.DMA,
          pltpu.SemaphoreType.DMA,
      ],
  )
  def k(xp_h, srcR_h, dstR_h, z8_h, out_h, acc, sbuf, dbuf, rows, gsem, ssem):
    c = lax.axis_index("c")
    s = lax.axis_index("s")
    pltpu.sync_copy(z8_h, acc.at[pl.ds(s * _ZR, _ZR)])
    plsc.subcore_barrier()
    n_outer = (_ER // 2) // 16 // _K
    base = c * (_ER // 2) + s * (n_outer * _K)
    _edge_loop(xp_h, srcR_h, dstR_h, acc, sbuf, dbuf, rows, gsem, ssem,
               base, n_outer)
    plsc.subcore_barrier()
    pltpu.sync_copy(acc.at[pl.ds(s * _ZR, _ZR)],
                    out_h.at[c].at[pl.ds(s * _ZR, _ZR)])

  return k(xp, srcR, dstR, z8)


def _sc_layer2(h0, h1, h2, h3, srcR, dstR, z32):
  @functools.partial(
      pl.kernel,
      out_type=jax.ShapeDtypeStruct((4, _NPAD, 32), jnp.float32),
      mesh=_sc_mesh(),
      compiler_params=pltpu.CompilerParams(use_tc_tiling_on_sc=False),
      scratch_types=[
          pltpu.VMEM_SHARED((_NPAD, 32), jnp.float32),
          pltpu.VMEM((_K, 128), jnp.int32),
          pltpu.VMEM((_K, 128), jnp.int32),
          pltpu.VMEM((_K, 128, 32), jnp.float32),
          pltpu.SemaphoreType.DMA,
          pltpu.SemaphoreType.DMA,
      ],
  )
  def k(h0_h, h1_h, h2_h, h3_h, srcR_h, dstR_h, z32_h, out_h,
        acc, sbuf, dbuf, rows, gsem, ssem):
    c = lax.axis_index("c")
    s = lax.axis_index("s")
    n_outer = _ER // 16 // _K

    def run_pass(tbl, q):
      pltpu.sync_copy(z32_h, acc.at[pl.ds(s * _ZR, _ZR)])
      plsc.subcore_barrier()
      _edge_loop(tbl, srcR_h, dstR_h, acc, sbuf, dbuf, rows, gsem, ssem,
                 s * (n_outer * _K), n_outer)
      plsc.subcore_barrier()
      pltpu.sync_copy(acc.at[pl.ds(s * _ZR, _ZR)],
                      out_h.at[q].at[pl.ds(s * _ZR, _ZR)])
      plsc.subcore_barrier()

    @pl.when(c == 0)
    def _():
      run_pass(h0_h, 0)
      run_pass(h1_h, 1)

    @pl.when(c == 1)
    def _():
      run_pass(h2_h, 2)
      run_pass(h3_h, 3)

  return k(h0, h1, h2, h3, srcR, dstR, z32)


def _ln(h, g, b):
  mu = jnp.mean(h, axis=-1, keepdims=True)
  var = jnp.mean((h - mu) ** 2, axis=-1, keepdims=True)
  return (h - mu) * lax.rsqrt(var + 1e-5) * g + b


def _tc_layer1(msum1, xp, wl, wr, b1, g1, be1):
  def body(ms_ref, xp_ref, wl_ref, wr_ref, b_ref, g_ref, be_ref,
           o0, o1, o2, o3):
    sm = ms_ref[0] + ms_ref[1]
    r = 1.0 / jnp.maximum(sm[:, 6:7], 1.0)
    mean8 = sm * r
    lin = jnp.dot(mean8, wl_ref[...], preferred_element_type=jnp.float32, precision=lax.Precision.HIGHEST)
    lin = lin + jnp.dot(xp_ref[...], wr_ref[...],
                        preferred_element_type=jnp.float32, precision=lax.Precision.HIGHEST)
    lin = lin + b_ref[...]
    h = jnp.maximum(_ln(lin, g_ref[...], be_ref[...]), 0.0)
    o0[...] = h[:, 0:32]
    o1[...] = h[:, 32:64]
    o2[...] = h[:, 64:96]
    o3[...] = h[:, 96:128]

  full = lambda shape: pl.BlockSpec(shape, lambda i: (0,) * len(shape))
  outs = [jax.ShapeDtypeStruct((_NTC, 32), jnp.float32)] * 4
  return pl.pallas_call(
      body,
      grid=(_NB,),
      in_specs=[
          pl.BlockSpec((2, _BN, 8), lambda i: (0, i, 0)),
          pl.BlockSpec((_BN, 8), lambda i: (i, 0)),
          full((8, 128)), full((8, 128)),
          full((1, 128)), full((1, 128)), full((1, 128)),
      ],
      out_specs=[pl.BlockSpec((_BN, 32), lambda i: (i, 0))] * 4,
      out_shape=outs,
  )(msum1, xp, wl, wr, b1, g1, be1)


def _tc_final(msum2, msum1, c0, c1, c2, c3, crd, bt, w2l, w2r, b2, g2, be2,
              bf, fcw, fcb2, fg, fb):
  def body(ms2, ms1, c0r, c1r, c2r, c3r, crdr, btr, w2lr, w2rr, b2r, g2r,
           be2r, bfr, fcwr, fcbr, fgr, fbr, out, acc):
    i = pl.program_id(0)

    @pl.when(i == 0)
    def _():
      acc[...] = jnp.zeros_like(acc)

    sm1 = ms1[0] + ms1[1]
    r = 1.0 / jnp.maximum(sm1[:, 6:7], 1.0)
    msum2b = jnp.concatenate([ms2[0], ms2[1], ms2[2], ms2[3]], axis=1)
    mean2 = msum2b * r
    h1 = jnp.concatenate([c0r[...], c1r[...], c2r[...], c3r[...]], axis=1)
    lin = jnp.dot(mean2, w2lr[...], preferred_element_type=jnp.float32, precision=lax.Precision.HIGHEST)
    lin = lin + jnp.dot(h1, w2rr[...], preferred_element_type=jnp.float32, precision=lax.Precision.HIGHEST)
    lin = lin + b2r[...]
    h2 = jnp.maximum(_ln(lin, g2r[...], be2r[...]), 0.0)

    crd_blk = crdr[...]
    parts = [h2]
    for sfac in _SCALES:
      ps = jnp.dot(crd_blk * sfac, bfr[...], preferred_element_type=jnp.float32)
      parts.append(jnp.sin(ps))
      parts.append(jnp.cos(ps))
    mk = (lax.broadcasted_iota(jnp.int32, (_BN, 128), 1) == 0)
    parts.append(mk.astype(jnp.float32))
    hcat = jnp.concatenate(parts, axis=1)

    bvec = btr[0, 0, :]
    oneh = (lax.broadcasted_iota(jnp.int32, (_G, _BN), 0)
            == bvec[None, :]).astype(jnp.float32)
    acc[...] += jnp.dot(oneh, hcat, preferred_element_type=jnp.float32, precision=lax.Precision.HIGHEST)

    @pl.when(i == _NB - 1)
    def _():
      a = acc[...]
      gcnt = jnp.maximum(a[:, 1152:1153], 1.0)
      g = a[:, :1152] * (1.0 / gcnt)
      o = jnp.dot(g, fcwr[...], preferred_element_type=jnp.float32, precision=lax.Precision.HIGHEST)
      o = o + fcbr[...]
      out[...] = _ln(o, fgr[...], fbr[...])

  full = lambda shape: pl.BlockSpec(shape, lambda i: (0,) * len(shape))
  return pl.pallas_call(
      body,
      grid=(_NB,),
      in_specs=[
          pl.BlockSpec((4, _BN, 32), lambda i: (0, i, 0)),
          pl.BlockSpec((2, _BN, 8), lambda i: (0, i, 0)),
          pl.BlockSpec((_BN, 32), lambda i: (i, 0)),
          pl.BlockSpec((_BN, 32), lambda i: (i, 0)),
          pl.BlockSpec((_BN, 32), lambda i: (i, 0)),
          pl.BlockSpec((_BN, 32), lambda i: (i, 0)),
          pl.BlockSpec((_BN, 4), lambda i: (i, 0)),
          pl.BlockSpec((1, 1, _BN), lambda i: (i, 0, 0)),
          full((128, 128)), full((128, 128)),
          full((1, 128)), full((1, 128)), full((1, 128)),
          full((4, 128)),
          full((1152, 256)), full((1, 256)), full((1, 256)), full((1, 256)),
      ],
      out_specs=pl.BlockSpec((_G, 256), lambda i: (0, 0)),
      out_shape=jax.ShapeDtypeStruct((_G, 256), jnp.float32),
      scratch_shapes=[pltpu.VMEM((_G, 1280), jnp.float32)],
  )(msum2, msum1, c0, c1, c2, c3, crd, bt, w2l, w2r, b2, g2, be2, bf, fcw,
    fcb2, fg, fb)


def kernel(x, edge_index, node_coords, batch, W1l, b1l, W1r, W2l, b2l, W2r,
           ln1g, ln1b, ln2g, ln2b, Bffm, fcW, fcb, flng, flnb):
  f32 = jnp.float32
  src = edge_index[0]
  dst = edge_index[1]
  npad = _EPAD - _E
  pidx = lax.iota(jnp.int32, npad) % 16
  srcR = jnp.concatenate([src, pidx]).reshape(_ER, 128)
  dstR = jnp.concatenate([dst, _N + pidx]).reshape(_ER, 128)

  xp = jnp.concatenate(
      [x, jnp.ones((_N, 1), f32), jnp.zeros((_N, 1), f32)], axis=1)
  xp = jnp.pad(xp, ((0, _NTC - _N), (0, 0)))
  z8 = jnp.zeros((_ZR, 8), f32)
  z32 = jnp.zeros((_ZR, 32), f32)

  msum1 = _sc_layer1(xp, srcR, dstR, z8)

  wl1 = jnp.pad(W1l.T, ((0, 2), (0, 0)))
  wr1 = jnp.pad(W1r.T, ((0, 2), (0, 0)))
  h1c = _tc_layer1(msum1, xp, wl1, wr1, b1l.reshape(1, 128),
                   ln1g.reshape(1, 128), ln1b.reshape(1, 128))

  msum2 = _sc_layer2(*h1c, srcR, dstR, z32)

  crd = jnp.pad(node_coords, ((0, _NTC - _N), (0, 1)))
  bt = jnp.pad(batch, (0, _NTC - _N), constant_values=_G).reshape(_NB, 1, _BN)
  bf = jnp.pad(Bffm, ((0, 1), (0, 0)))
  out = _tc_final(msum2, msum1, *h1c, crd, bt, W2l.T, W2r.T,
                  b2l.reshape(1, 128), ln2g.reshape(1, 128),
                  ln2b.reshape(1, 128), bf, fcW.T, fcb.reshape(1, 256),
                  flng.reshape(1, 256), flnb.reshape(1, 256))
  return out


# speedup vs baseline: 6.2861x; 1.3563x over previous
import functools

import jax
import jax.numpy as jnp
from jax import lax
from jax.experimental import pallas as pl
from jax.experimental.pallas import tpu as pltpu
from jax.experimental.pallas import tpu_sc as plsc

_N = 50000
_E = 800000
_G = 64
_BN = 256
_NB = 196
_NTC = _NB * _BN
_NPAD = 51200
_ZR = _NPAD // 16
_ER = 6400
_EPAD = _ER * 128
_K = 4
_SCALES = (1.0, 5.0, 10.0, 30.0)


def _sc_mesh():
  return plsc.VectorSubcoreMesh(
      core_axis_name="c", subcore_axis_name="s", num_cores=2, num_subcores=16)


def _edge_loop(gather_fn, srcR, dstR, acc, sbuf, dbuf, rows, gsem, ssem,
               base_row, n_outer):
  def outer(i, carry):
    row0 = base_row + i * _K
    pltpu.sync_copy(srcR.at[pl.ds(row0, _K)], sbuf)
    pltpu.sync_copy(dstR.at[pl.ds(row0, _K)], dbuf)
    gd = [gather_fn(sbuf.at[j], rows.at[j], gsem) for j in range(_K)]
    for d in gd:
      d.wait()
    sd = [pltpu.async_copy(rows.at[j], acc.at[dbuf.at[j]], ssem, add=True)
          for j in range(_K)]
    for d in sd:
      d.wait()
    return carry

  lax.fori_loop(0, n_outer, outer, 0)


def _sc_layer1(xp, srcR, dstR, z8):
  @functools.partial(
      pl.kernel,
      out_type=jax.ShapeDtypeStruct((_NPAD, 16), jnp.float32),
      mesh=_sc_mesh(),
      compiler_params=pltpu.CompilerParams(use_tc_tiling_on_sc=False),
      scratch_types=[
          pltpu.VMEM_SHARED((_NPAD, 8), jnp.float32),
          pltpu.VMEM((_K, 128), jnp.int32),
          pltpu.VMEM((_K, 128), jnp.int32),
          pltpu.VMEM((_K, 128, 8), jnp.float32),
          pltpu.SemaphoreType.DMA,
          pltpu.SemaphoreType.DMA,
      ],
  )
  def k(xp_h, srcR_h, dstR_h, z8_h, out_h, acc, sbuf, dbuf, rows, gsem, ssem):
    c = lax.axis_index("c")
    s = lax.axis_index("s")
    pltpu.sync_copy(z8_h, acc.at[pl.ds(s * _ZR, _ZR)])
    plsc.subcore_barrier()
    n_outer = (_ER // 2) // 16 // _K
    base = c * (_ER // 2) + s * (n_outer * _K)
    gather = lambda idx, dst, sem: pltpu.async_copy(xp_h.at[idx], dst, sem)
    _edge_loop(gather, srcR_h, dstR_h, acc, sbuf, dbuf, rows, gsem, ssem,
               base, n_outer)
    plsc.subcore_barrier()

    @pl.when(c == 0)
    def _():
      pltpu.sync_copy(acc.at[pl.ds(s * _ZR, _ZR)],
                      out_h.at[pl.ds(s * _ZR, _ZR), pl.ds(0, 8)])

    @pl.when(c == 1)
    def _():
      pltpu.sync_copy(acc.at[pl.ds(s * _ZR, _ZR)],
                      out_h.at[pl.ds(s * _ZR, _ZR), pl.ds(8, 8)])

  return k(xp, srcR, dstR, z8)


_TROWS = _NTC // 16
_TSTEP = 112


def _sc_layer2(h1, srcR, dstR, z32):
  @functools.partial(
      pl.kernel,
      out_type=(jax.ShapeDtypeStruct((_NPAD, 128), jnp.float32),
                jax.ShapeDtypeStruct((4, _NTC, 32), jnp.float32)),
      mesh=_sc_mesh(),
      compiler_params=pltpu.CompilerParams(use_tc_tiling_on_sc=False),
      scratch_types=[
          pltpu.VMEM_SHARED((_NPAD, 32), jnp.float32),
          pltpu.VMEM((_K, 128), jnp.int32),
          pltpu.VMEM((_K, 128), jnp.int32),
          pltpu.VMEM((_K, 128, 32), jnp.float32),
          pltpu.VMEM((_TSTEP, 32), jnp.float32),
          pltpu.SemaphoreType.DMA,
          pltpu.SemaphoreType.DMA,
      ],
  )
  def k(h1_h, srcR_h, dstR_h, z32_h, out_h, tbl_h,
        acc, sbuf, dbuf, rows, stage, gsem, ssem):
    c = lax.axis_index("c")
    s = lax.axis_index("s")
    n_outer = _ER // 16 // _K

    def split(qj):
      def it(i, carry):
        r0 = s * _TROWS + i * _TSTEP
        pltpu.sync_copy(h1_h.at[pl.ds(r0, _TSTEP), pl.ds(qj * 32, 32)], stage)
        pltpu.sync_copy(stage, tbl_h.at[qj].at[pl.ds(r0, _TSTEP)])
        return carry
      lax.fori_loop(0, _TROWS // _TSTEP, it, 0)

    def run_pass(q):
      pltpu.sync_copy(z32_h, acc.at[pl.ds(s * _ZR, _ZR)])
      plsc.subcore_barrier()
      gather = lambda idx, dst, sem: pltpu.async_copy(
          tbl_h.at[q].at[idx], dst, sem)
      _edge_loop(gather, srcR_h, dstR_h, acc, sbuf, dbuf, rows, gsem, ssem,
                 s * (n_outer * _K), n_outer)
      plsc.subcore_barrier()
      pltpu.sync_copy(acc.at[pl.ds(s * _ZR, _ZR)],
                      out_h.at[pl.ds(s * _ZR, _ZR), pl.ds(q * 32, 32)])
      plsc.subcore_barrier()

    @pl.when(c == 0)
    def _():
      split(0)
      split(1)
      run_pass(0)
      run_pass(1)

    @pl.when(c == 1)
    def _():
      split(2)
      split(3)
      run_pass(2)
      run_pass(3)

  return k(h1, srcR, dstR, z32)[0]


def _ln(h, g, b):
  mu = jnp.mean(h, axis=-1, keepdims=True)
  var = jnp.mean((h - mu) ** 2, axis=-1, keepdims=True)
  return (h - mu) * lax.rsqrt(var + 1e-5) * g + b


_HI = lax.Precision.HIGHEST


def _tc_layer1(msum1, xp, wl, wr, b1, g1, be1):
  def body(ms_ref, xp_ref, wl_ref, wr_ref, b_ref, g_ref, be_ref, o_ref):
    sm = ms_ref[:, 0:8] + ms_ref[:, 8:16]
    r = 1.0 / jnp.maximum(sm[:, 6:7], 1.0)
    mean8 = sm * r
    lin = jnp.dot(mean8, wl_ref[...], preferred_element_type=jnp.float32,
                  precision=_HI)
    lin = lin + jnp.dot(xp_ref[...], wr_ref[...],
                        preferred_element_type=jnp.float32, precision=_HI)
    lin = lin + b_ref[...]
    o_ref[...] = jnp.maximum(_ln(lin, g_ref[...], be_ref[...]), 0.0)

  full = lambda shape: pl.BlockSpec(shape, lambda i: (0,) * len(shape))
  return pl.pallas_call(
      body,
      grid=(_NB,),
      in_specs=[
          pl.BlockSpec((_BN, 16), lambda i: (i, 0)),
          pl.BlockSpec((_BN, 8), lambda i: (i, 0)),
          full((8, 128)), full((8, 128)),
          full((1, 128)), full((1, 128)), full((1, 128)),
      ],
      out_specs=pl.BlockSpec((_BN, 128), lambda i: (i, 0)),
      out_shape=jax.ShapeDtypeStruct((_NTC, 128), jnp.float32),
  )(msum1, xp, wl, wr, b1, g1, be1)


def _tc_ffm(crd, bt, bf):
  def body(crdr, btr, bfr, out, acc):
    i = pl.program_id(0)

    @pl.when(i == 0)
    def _():
      acc[...] = jnp.zeros_like(acc)

    crd_blk = crdr[...]
    parts = []
    for sfac in _SCALES:
      ps = jnp.dot(crd_blk * sfac, bfr[...], preferred_element_type=jnp.float32)
      parts.append(jnp.sin(ps))
      parts.append(jnp.cos(ps))
    mk = (lax.broadcasted_iota(jnp.int32, (_BN, 128), 1) == 0)
    parts.append(mk.astype(jnp.float32))
    hcat = jnp.concatenate(parts, axis=1)

    bvec = btr[0, 0, :]
    oneh = (lax.broadcasted_iota(jnp.int32, (_G, _BN), 0)
            == bvec[None, :]).astype(jnp.float32)
    acc[...] += jnp.dot(oneh, hcat, preferred_element_type=jnp.float32,
                        precision=_HI)

    @pl.when(i == _NB - 1)
    def _():
      out[...] = acc[...]

  full = lambda shape: pl.BlockSpec(shape, lambda i: (0,) * len(shape))
  return pl.pallas_call(
      body,
      grid=(_NB,),
      in_specs=[
          pl.BlockSpec((_BN, 4), lambda i: (i, 0)),
          pl.BlockSpec((1, 1, _BN), lambda i: (i, 0, 0)),
          full((4, 128)),
      ],
      out_specs=pl.BlockSpec((_G, 1152), lambda i: (0, 0)),
      out_shape=jax.ShapeDtypeStruct((_G, 1152), jnp.float32),
      scratch_shapes=[pltpu.VMEM((_G, 1152), jnp.float32)],
  )(crd, bt, bf)


def _tc_final(msum2, msum1, h1, bt, ffmp, w2l, w2r, b2, g2, be2,
              fcw, fcb2, fg, fb):
  def body(ms2, ms1, h1r, btr, ffr, w2lr, w2rr, b2r, g2r, be2r, fcwr, fcbr,
           fgr, fbr, out, acc):
    i = pl.program_id(0)

    @pl.when(i == 0)
    def _():
      acc[...] = jnp.zeros_like(acc)

    sm1 = ms1[:, 0:8] + ms1[:, 8:16]
    r = 1.0 / jnp.maximum(sm1[:, 6:7], 1.0)
    mean2 = ms2[...] * r
    lin = jnp.dot(mean2, w2lr[...], preferred_element_type=jnp.float32,
                  precision=_HI)
    lin = lin + jnp.dot(h1r[...], w2rr[...], preferred_element_type=jnp.float32,
                        precision=_HI)
    lin = lin + b2r[...]
    h2 = jnp.maximum(_ln(lin, g2r[...], be2r[...]), 0.0)

    bvec = btr[0, 0, :]
    oneh = (lax.broadcasted_iota(jnp.int32, (_G, _BN), 0)
            == bvec[None, :]).astype(jnp.float32)
    acc[...] += jnp.dot(oneh, h2, preferred_element_type=jnp.float32,
                        precision=_HI)

    @pl.when(i == _NB - 1)
    def _():
      fp = ffr[...]
      rg = 1.0 / jnp.maximum(fp[:, 1024:1025], 1.0)
      g = jnp.concatenate([acc[...], fp[:, 0:1024]], axis=1) * rg
      o = jnp.dot(g, fcwr[...], preferred_element_type=jnp.float32,
                  precision=_HI)
      o = o + fcbr[...]
      out[...] = _ln(o, fgr[...], fbr[...])

  full = lambda shape: pl.BlockSpec(shape, lambda i: (0,) * len(shape))
  return pl.pallas_call(
      body,
      grid=(_NB,),
      in_specs=[
          pl.BlockSpec((_BN, 128), lambda i: (i, 0)),
          pl.BlockSpec((_BN, 16), lambda i: (i, 0)),
          pl.BlockSpec((_BN, 128), lambda i: (i, 0)),
          pl.BlockSpec((1, 1, _BN), lambda i: (i, 0, 0)),
          full((_G, 1152)),
          full((128, 128)), full((128, 128)),
          full((1, 128)), full((1, 128)), full((1, 128)),
          full((1152, 256)), full((1, 256)), full((1, 256)), full((1, 256)),
      ],
      out_specs=pl.BlockSpec((_G, 256), lambda i: (0, 0)),
      out_shape=jax.ShapeDtypeStruct((_G, 256), jnp.float32),
      scratch_shapes=[pltpu.VMEM((_G, 128), jnp.float32)],
  )(msum2, msum1, h1, bt, ffmp, w2l, w2r, b2, g2, be2, fcw, fcb2, fg, fb)


def kernel(x, edge_index, node_coords, batch, W1l, b1l, W1r, W2l, b2l, W2r,
           ln1g, ln1b, ln2g, ln2b, Bffm, fcW, fcb, flng, flnb):
  f32 = jnp.float32
  src = edge_index[0]
  dst = edge_index[1]
  npad = _EPAD - _E
  pidx = lax.iota(jnp.int32, npad) % 16
  srcR = jnp.concatenate([src, pidx]).reshape(_ER, 128)
  dstR = jnp.concatenate([dst, _N + pidx]).reshape(_ER, 128)

  xp = jnp.concatenate(
      [x, jnp.ones((_N, 1), f32), jnp.zeros((_N, 1), f32)], axis=1)
  xp = jnp.pad(xp, ((0, _NTC - _N), (0, 0)))
  z8 = jnp.zeros((_ZR, 8), f32)
  z32 = jnp.zeros((_ZR, 32), f32)

  msum1 = _sc_layer1(xp, srcR, dstR, z8)

  wl1 = jnp.pad(W1l.T, ((0, 2), (0, 0)))
  wr1 = jnp.pad(W1r.T, ((0, 2), (0, 0)))
  h1 = _tc_layer1(msum1, xp, wl1, wr1, b1l.reshape(1, 128),
                  ln1g.reshape(1, 128), ln1b.reshape(1, 128))

  msum2 = _sc_layer2(h1, srcR, dstR, z32)

  crd = jnp.pad(node_coords, ((0, _NTC - _N), (0, 1)))
  bt = jnp.pad(batch, (0, _NTC - _N), constant_values=_G).reshape(_NB, 1, _BN)
  bf = jnp.pad(Bffm, ((0, 1), (0, 0)))
  ffmp = _tc_ffm(crd, bt, bf)

  out = _tc_final(msum2, msum1, h1, bt, ffmp, W2l.T, W2r.T,
                  b2l.reshape(1, 128), ln2g.reshape(1, 128),
                  ln2b.reshape(1, 128), fcW.T, fcb.reshape(1, 256),
                  flng.reshape(1, 256), flnb.reshape(1, 256))
  return out


# speedup vs baseline: 7.3374x; 1.1672x over previous
import functools

import jax
import jax.numpy as jnp
from jax import lax
from jax.experimental import pallas as pl
from jax.experimental.pallas import tpu as pltpu
from jax.experimental.pallas import tpu_sc as plsc

_N = 50000
_E = 800000
_G = 64
_BN = 256
_NB = 196
_NTC = _NB * _BN
_NPAD = 51200
_ZR = _NPAD // 16
_ER = 6400
_EPAD = _ER * 128
_K1 = 4
_K2 = 2
_SCALES = (1.0, 5.0, 10.0, 30.0)


def _sc_mesh():
  return plsc.VectorSubcoreMesh(
      core_axis_name="c", subcore_axis_name="s", num_cores=2, num_subcores=16)


def _edge_loop(k, gather_fn, drain_src, srcR, dstR, acc, sbuf, dbuf, rows,
               gsems, ssems, base_row, n_groups):
  def stage_and_fire(g, grp):
    row0 = base_row + grp * k
    pltpu.sync_copy(srcR.at[pl.ds(row0, k)], sbuf.at[g])
    pltpu.sync_copy(dstR.at[pl.ds(row0, k)], dbuf.at[g])
    for j in range(k):
      gather_fn(sbuf.at[g].at[j], rows.at[g].at[j], gsems[g])

  def drain_and_scatter(g):
    for j in range(k):
      pltpu.make_async_copy(drain_src, rows.at[g].at[j], gsems[g]).wait()
    sd = [pltpu.async_copy(rows.at[g].at[j], acc.at[dbuf.at[g].at[j]],
                           ssems[g], add=True) for j in range(k)]
    for d in sd:
      d.wait()

  stage_and_fire(0, 0)
  stage_and_fire(1, 1)

  def body(i, carry):
    drain_and_scatter(0)
    stage_and_fire(0, 2 * i + 2)
    drain_and_scatter(1)
    stage_and_fire(1, 2 * i + 3)
    return carry

  lax.fori_loop(0, n_groups // 2 - 1, body, 0)
  drain_and_scatter(0)
  drain_and_scatter(1)


def _sc_layer1(xp, srcR, dstR, z8):
  @functools.partial(
      pl.kernel,
      out_type=jax.ShapeDtypeStruct((_NPAD, 16), jnp.float32),
      mesh=_sc_mesh(),
      compiler_params=pltpu.CompilerParams(use_tc_tiling_on_sc=False),
      scratch_types=[
          pltpu.VMEM_SHARED((_NPAD, 8), jnp.float32),
          pltpu.VMEM((2, _K1, 128), jnp.int32),
          pltpu.VMEM((2, _K1, 128), jnp.int32),
          pltpu.VMEM((2, _K1, 128, 8), jnp.float32),
          pltpu.SemaphoreType.DMA,
          pltpu.SemaphoreType.DMA,
          pltpu.SemaphoreType.DMA,
          pltpu.SemaphoreType.DMA,
      ],
  )
  def k(xp_h, srcR_h, dstR_h, z8_h, out_h, acc, sbuf, dbuf, rows,
        gsem0, gsem1, ssem0, ssem1):
    c = lax.axis_index("c")
    s = lax.axis_index("s")
    pltpu.sync_copy(z8_h, acc.at[pl.ds(s * _ZR, _ZR)])
    plsc.subcore_barrier()
    n_groups = (_ER // 2) // 16 // _K1
    base = c * (_ER // 2) + s * (n_groups * _K1)
    gather = lambda idx, dst, sem: pltpu.async_copy(xp_h.at[idx], dst, sem)
    _edge_loop(_K1, gather, xp_h.at[pl.ds(0, 128)], srcR_h, dstR_h, acc,
               sbuf, dbuf, rows, (gsem0, gsem1), (ssem0, ssem1),
               base, n_groups)
    plsc.subcore_barrier()

    @pl.when(c == 0)
    def _():
      pltpu.sync_copy(acc.at[pl.ds(s * _ZR, _ZR)],
                      out_h.at[pl.ds(s * _ZR, _ZR), pl.ds(0, 8)])

    @pl.when(c == 1)
    def _():
      pltpu.sync_copy(acc.at[pl.ds(s * _ZR, _ZR)],
                      out_h.at[pl.ds(s * _ZR, _ZR), pl.ds(8, 8)])

  return k(xp, srcR, dstR, z8)


_TROWS = _NTC // 16
_TSTEP = 112


def _sc_layer2(h1, srcR, dstR, z32):
  @functools.partial(
      pl.kernel,
      out_type=(jax.ShapeDtypeStruct((_NPAD, 128), jnp.float32),
                jax.ShapeDtypeStruct((4, _NTC, 32), jnp.float32)),
      mesh=_sc_mesh(),
      compiler_params=pltpu.CompilerParams(use_tc_tiling_on_sc=False),
      scratch_types=[
          pltpu.VMEM_SHARED((_NPAD, 32), jnp.float32),
          pltpu.VMEM((2, _K2, 128), jnp.int32),
          pltpu.VMEM((2, _K2, 128), jnp.int32),
          pltpu.VMEM((2, _K2, 128, 32), jnp.float32),
          pltpu.VMEM((_TSTEP, 32), jnp.float32),
          pltpu.SemaphoreType.DMA,
          pltpu.SemaphoreType.DMA,
          pltpu.SemaphoreType.DMA,
          pltpu.SemaphoreType.DMA,
      ],
  )
  def k(h1_h, srcR_h, dstR_h, z32_h, out_h, tbl_h,
        acc, sbuf, dbuf, rows, stage, gsem0, gsem1, ssem0, ssem1):
    c = lax.axis_index("c")
    s = lax.axis_index("s")
    n_groups = _ER // 16 // _K2

    def split(qj):
      def it(i, carry):
        r0 = s * _TROWS + i * _TSTEP
        pltpu.sync_copy(h1_h.at[pl.ds(r0, _TSTEP), pl.ds(qj * 32, 32)], stage)
        pltpu.sync_copy(stage, tbl_h.at[qj].at[pl.ds(r0, _TSTEP)])
        return carry
      lax.fori_loop(0, _TROWS // _TSTEP, it, 0)

    def run_pass(q):
      pltpu.sync_copy(z32_h, acc.at[pl.ds(s * _ZR, _ZR)])
      plsc.subcore_barrier()
      gather = lambda idx, dst, sem: pltpu.async_copy(
          tbl_h.at[q].at[idx], dst, sem)
      _edge_loop(_K2, gather, tbl_h.at[q].at[pl.ds(0, 128)], srcR_h, dstR_h,
                 acc, sbuf, dbuf, rows, (gsem0, gsem1), (ssem0, ssem1),
                 s * (n_groups * _K2), n_groups)
      plsc.subcore_barrier()
      pltpu.sync_copy(acc.at[pl.ds(s * _ZR, _ZR)],
                      out_h.at[pl.ds(s * _ZR, _ZR), pl.ds(q * 32, 32)])
      plsc.subcore_barrier()

    @pl.when(c == 0)
    def _():
      split(0)
      split(1)
      run_pass(0)
      run_pass(1)

    @pl.when(c == 1)
    def _():
      split(2)
      split(3)
      run_pass(2)
      run_pass(3)

  return k(h1, srcR, dstR, z32)[0]


def _ln(h, g, b):
  mu = jnp.mean(h, axis=-1, keepdims=True)
  var = jnp.mean((h - mu) ** 2, axis=-1, keepdims=True)
  return (h - mu) * lax.rsqrt(var + 1e-5) * g + b


_HI = lax.Precision.HIGHEST


def _tc_layer1(msum1, xp, wl, wr, b1, g1, be1):
  def body(ms_ref, xp_ref, wl_ref, wr_ref, b_ref, g_ref, be_ref, o_ref):
    sm = ms_ref[:, 0:8] + ms_ref[:, 8:16]
    r = 1.0 / jnp.maximum(sm[:, 6:7], 1.0)
    mean8 = sm * r
    lin = jnp.dot(mean8, wl_ref[...], preferred_element_type=jnp.float32,
                  precision=_HI)
    lin = lin + jnp.dot(xp_ref[...], wr_ref[...],
                        preferred_element_type=jnp.float32, precision=_HI)
    lin = lin + b_ref[...]
    o_ref[...] = jnp.maximum(_ln(lin, g_ref[...], be_ref[...]), 0.0)

  full = lambda shape: pl.BlockSpec(shape, lambda i: (0,) * len(shape))
  return pl.pallas_call(
      body,
      grid=(_NB,),
      in_specs=[
          pl.BlockSpec((_BN, 16), lambda i: (i, 0)),
          pl.BlockSpec((_BN, 8), lambda i: (i, 0)),
          full((8, 128)), full((8, 128)),
          full((1, 128)), full((1, 128)), full((1, 128)),
      ],
      out_specs=pl.BlockSpec((_BN, 128), lambda i: (i, 0)),
      out_shape=jax.ShapeDtypeStruct((_NTC, 128), jnp.float32),
  )(msum1, xp, wl, wr, b1, g1, be1)


def _tc_ffm(crd, bt, bf):
  def body(crdr, btr, bfr, out, acc):
    i = pl.program_id(0)

    @pl.when(i == 0)
    def _():
      acc[...] = jnp.zeros_like(acc)

    crd_blk = crdr[...]
    parts = []
    for sfac in _SCALES:
      ps = jnp.dot(crd_blk * sfac, bfr[...], preferred_element_type=jnp.float32)
      parts.append(jnp.sin(ps))
      parts.append(jnp.cos(ps))
    mk = (lax.broadcasted_iota(jnp.int32, (_BN, 128), 1) == 0)
    parts.append(mk.astype(jnp.float32))
    hcat = jnp.concatenate(parts, axis=1)

    bvec = btr[0, 0, :]
    oneh = (lax.broadcasted_iota(jnp.int32, (_G, _BN), 0)
            == bvec[None, :]).astype(jnp.float32)
    acc[...] += jnp.dot(oneh, hcat, preferred_element_type=jnp.float32,
                        precision=_HI)

    @pl.when(i == _NB - 1)
    def _():
      out[...] = acc[...]

  full = lambda shape: pl.BlockSpec(shape, lambda i: (0,) * len(shape))
  return pl.pallas_call(
      body,
      grid=(_NB,),
      in_specs=[
          pl.BlockSpec((_BN, 4), lambda i: (i, 0)),
          pl.BlockSpec((1, 1, _BN), lambda i: (i, 0, 0)),
          full((4, 128)),
      ],
      out_specs=pl.BlockSpec((_G, 1152), lambda i: (0, 0)),
      out_shape=jax.ShapeDtypeStruct((_G, 1152), jnp.float32),
      scratch_shapes=[pltpu.VMEM((_G, 1152), jnp.float32)],
  )(crd, bt, bf)


def _tc_final(msum2, msum1, h1, bt, ffmp, w2l, w2r, b2, g2, be2,
              fcw, fcb2, fg, fb):
  def body(ms2, ms1, h1r, btr, ffr, w2lr, w2rr, b2r, g2r, be2r, fcwr, fcbr,
           fgr, fbr, out, acc):
    i = pl.program_id(0)

    @pl.when(i == 0)
    def _():
      acc[...] = jnp.zeros_like(acc)

    sm1 = ms1[:, 0:8] + ms1[:, 8:16]
    r = 1.0 / jnp.maximum(sm1[:, 6:7], 1.0)
    mean2 = ms2[...] * r
    lin = jnp.dot(mean2, w2lr[...], preferred_element_type=jnp.float32,
                  precision=_HI)
    lin = lin + jnp.dot(h1r[...], w2rr[...], preferred_element_type=jnp.float32,
                        precision=_HI)
    lin = lin + b2r[...]
    h2 = jnp.maximum(_ln(lin, g2r[...], be2r[...]), 0.0)

    bvec = btr[0, 0, :]
    oneh = (lax.broadcasted_iota(jnp.int32, (_G, _BN), 0)
            == bvec[None, :]).astype(jnp.float32)
    acc[...] += jnp.dot(oneh, h2, preferred_element_type=jnp.float32,
                        precision=_HI)

    @pl.when(i == _NB - 1)
    def _():
      fp = ffr[...]
      rg = 1.0 / jnp.maximum(fp[:, 1024:1025], 1.0)
      g = jnp.concatenate([acc[...], fp[:, 0:1024]], axis=1) * rg
      o = jnp.dot(g, fcwr[...], preferred_element_type=jnp.float32,
                  precision=_HI)
      o = o + fcbr[...]
      out[...] = _ln(o, fgr[...], fbr[...])

  full = lambda shape: pl.BlockSpec(shape, lambda i: (0,) * len(shape))
  return pl.pallas_call(
      body,
      grid=(_NB,),
      in_specs=[
          pl.BlockSpec((_BN, 128), lambda i: (i, 0)),
          pl.BlockSpec((_BN, 16), lambda i: (i, 0)),
          pl.BlockSpec((_BN, 128), lambda i: (i, 0)),
          pl.BlockSpec((1, 1, _BN), lambda i: (i, 0, 0)),
          full((_G, 1152)),
          full((128, 128)), full((128, 128)),
          full((1, 128)), full((1, 128)), full((1, 128)),
          full((1152, 256)), full((1, 256)), full((1, 256)), full((1, 256)),
      ],
      out_specs=pl.BlockSpec((_G, 256), lambda i: (0, 0)),
      out_shape=jax.ShapeDtypeStruct((_G, 256), jnp.float32),
      scratch_shapes=[pltpu.VMEM((_G, 128), jnp.float32)],
  )(msum2, msum1, h1, bt, ffmp, w2l, w2r, b2, g2, be2, fcw, fcb2, fg, fb)


def kernel(x, edge_index, node_coords, batch, W1l, b1l, W1r, W2l, b2l, W2r,
           ln1g, ln1b, ln2g, ln2b, Bffm, fcW, fcb, flng, flnb):
  f32 = jnp.float32
  src = edge_index[0]
  dst = edge_index[1]
  npad = _EPAD - _E
  pidx = lax.iota(jnp.int32, npad) % 16
  srcR = jnp.concatenate([src, pidx]).reshape(_ER, 128)
  dstR = jnp.concatenate([dst, _N + pidx]).reshape(_ER, 128)

  xp = jnp.concatenate(
      [x, jnp.ones((_N, 1), f32), jnp.zeros((_N, 1), f32)], axis=1)
  xp = jnp.pad(xp, ((0, _NTC - _N), (0, 0)))
  z8 = jnp.zeros((_ZR, 8), f32)
  z32 = jnp.zeros((_ZR, 32), f32)

  msum1 = _sc_layer1(xp, srcR, dstR, z8)

  wl1 = jnp.pad(W1l.T, ((0, 2), (0, 0)))
  wr1 = jnp.pad(W1r.T, ((0, 2), (0, 0)))
  h1 = _tc_layer1(msum1, xp, wl1, wr1, b1l.reshape(1, 128),
                  ln1g.reshape(1, 128), ln1b.reshape(1, 128))

  msum2 = _sc_layer2(h1, srcR, dstR, z32)

  crd = jnp.pad(node_coords, ((0, _NTC - _N), (0, 1)))
  bt = jnp.pad(batch, (0, _NTC - _N), constant_values=_G).reshape(_NB, 1, _BN)
  bf = jnp.pad(Bffm, ((0, 1), (0, 0)))
  ffmp = _tc_ffm(crd, bt, bf)

  out = _tc_final(msum2, msum1, h1, bt, ffmp, W2l.T, W2r.T,
                  b2l.reshape(1, 128), ln2g.reshape(1, 128),
                  ln2b.reshape(1, 128), fcW.T, fcb.reshape(1, 256),
                  flng.reshape(1, 256), flnb.reshape(1, 256))
  return out


# speedup vs baseline: 8.1743x; 1.1141x over previous
import functools

import jax
import jax.numpy as jnp
from jax import lax
from jax.experimental import pallas as pl
from jax.experimental.pallas import tpu as pltpu
from jax.experimental.pallas import tpu_sc as plsc

_N = 50000
_E = 800000
_G = 64
_BN = 256
_NB = 196
_NTC = _NB * _BN
_NPAD = 51200
_ZR = _NPAD // 16
_ER = 6400
_ERX = _ER + 64
_EPAD = _ER * 128
_K1 = 4
_G1 = 5
_B1 = 10
_K2 = 2
_G2 = 10
_B2 = 20
_SCALES = (1.0, 5.0, 10.0, 30.0)


def _sc_mesh():
  return plsc.VectorSubcoreMesh(
      core_axis_name="c", subcore_axis_name="s", num_cores=2, num_subcores=16)


def _edge_loop(k, gpb, n_blocks, gather_fn, drain_src, srcR, dstR, acc,
               isrc, idst, rows, gsems, ssems, base_row):
  rpb = gpb * k

  def stage(blk, ib):
    row0 = base_row + blk * rpb
    pltpu.sync_copy(srcR.at[pl.ds(row0, rpb)], isrc.at[ib])
    pltpu.sync_copy(dstR.at[pl.ds(row0, rpb)], idst.at[ib])

  pend = [None, None]

  def fire(g, ib, t):
    for j in range(k):
      gather_fn(isrc.at[ib].at[t * k + j], rows.at[g].at[j], gsems[g])
    pend[g] = (ib, t)

  def finish(g):
    ib, t = pend[g]
    for j in range(k):
      pltpu.make_async_copy(drain_src, rows.at[g].at[j], gsems[g]).wait()
    sd = [pltpu.async_copy(rows.at[g].at[j], acc.at[idst.at[ib].at[t * k + j]],
                           ssems[g], add=True) for j in range(k)]
    for d in sd:
      d.wait()

  def run_block(ib, next_blk):
    for t in range(2, gpb):
      g = t % 2
      finish(g)
      fire(g, ib, t)
    if next_blk is not None:
      stage(next_blk, 1 - ib)
      for t in range(2):
        finish(t)
        fire(t, 1 - ib, t)

  stage(0, 0)
  fire(0, 0, 0)
  fire(1, 0, 1)

  def pair(i, carry):
    run_block(0, 2 * i + 1)
    run_block(1, 2 * i + 2)
    return carry

  lax.fori_loop(0, n_blocks // 2 - 1, pair, 0)
  run_block(0, n_blocks - 1)
  run_block(1, None)
  finish(0)
  finish(1)


def _sc_layer1(xp, srcR, dstR, z8):
  @functools.partial(
      pl.kernel,
      out_type=jax.ShapeDtypeStruct((_NPAD, 16), jnp.float32),
      mesh=_sc_mesh(),
      compiler_params=pltpu.CompilerParams(use_tc_tiling_on_sc=False),
      scratch_types=[
          pltpu.VMEM_SHARED((_NPAD, 8), jnp.float32),
          pltpu.VMEM((2, _G1 * _K1, 128), jnp.int32),
          pltpu.VMEM((2, _G1 * _K1, 128), jnp.int32),
          pltpu.VMEM((2, _K1, 128, 8), jnp.float32),
          pltpu.SemaphoreType.DMA,
          pltpu.SemaphoreType.DMA,
          pltpu.SemaphoreType.DMA,
          pltpu.SemaphoreType.DMA,
      ],
  )
  def k(xp_h, srcR_h, dstR_h, z8_h, out_h, acc, isrc, idst, rows,
        gsem0, gsem1, ssem0, ssem1):
    c = lax.axis_index("c")
    s = lax.axis_index("s")
    pltpu.sync_copy(z8_h, acc.at[pl.ds(s * _ZR, _ZR)])
    plsc.subcore_barrier()
    rows_per_tile = (_ER // 2) // 16
    base = c * (_ER // 2) + s * rows_per_tile
    gather = lambda idx, dst, sem: pltpu.async_copy(xp_h.at[idx], dst, sem)
    _edge_loop(_K1, _G1, _B1, gather, xp_h.at[pl.ds(0, 128)], srcR_h, dstR_h,
               acc, isrc, idst, rows, (gsem0, gsem1), (ssem0, ssem1), base)
    plsc.subcore_barrier()

    @pl.when(c == 0)
    def _():
      pltpu.sync_copy(acc.at[pl.ds(s * _ZR, _ZR)],
                      out_h.at[pl.ds(s * _ZR, _ZR), pl.ds(0, 8)])

    @pl.when(c == 1)
    def _():
      pltpu.sync_copy(acc.at[pl.ds(s * _ZR, _ZR)],
                      out_h.at[pl.ds(s * _ZR, _ZR), pl.ds(8, 8)])

  return k(xp, srcR, dstR, z8)


_TROWS = _NTC // 16
_TSTEP = 112


def _sc_layer2(h1, srcR, dstR, z32):
  @functools.partial(
      pl.kernel,
      out_type=(jax.ShapeDtypeStruct((_NPAD, 128), jnp.float32),
                jax.ShapeDtypeStruct((4, _NTC, 32), jnp.float32)),
      mesh=_sc_mesh(),
      compiler_params=pltpu.CompilerParams(use_tc_tiling_on_sc=False),
      scratch_types=[
          pltpu.VMEM_SHARED((_NPAD, 32), jnp.float32),
          pltpu.VMEM((2, _G2 * _K2, 128), jnp.int32),
          pltpu.VMEM((2, _G2 * _K2, 128), jnp.int32),
          pltpu.VMEM((2, _K2, 128, 32), jnp.float32),
          pltpu.SemaphoreType.DMA,
          pltpu.SemaphoreType.DMA,
          pltpu.SemaphoreType.DMA,
          pltpu.SemaphoreType.DMA,
      ],
  )
  def k(h1_h, srcR_h, dstR_h, z32_h, out_h, tbl_h,
        acc, isrc, idst, rows, gsem0, gsem1, ssem0, ssem1):
    c = lax.axis_index("c")
    s = lax.axis_index("s")

    def split(qj):
      stage = rows.at[0, 0, pl.ds(0, _TSTEP)]
      def it(i, carry):
        r0 = s * _TROWS + i * _TSTEP
        pltpu.sync_copy(h1_h.at[pl.ds(r0, _TSTEP), pl.ds(qj * 32, 32)], stage)
        pltpu.sync_copy(stage, tbl_h.at[qj].at[pl.ds(r0, _TSTEP)])
        return carry
      lax.fori_loop(0, _TROWS // _TSTEP, it, 0)

    def run_pass(q):
      pltpu.sync_copy(z32_h, acc.at[pl.ds(s * _ZR, _ZR)])
      plsc.subcore_barrier()
      gather = lambda idx, dst, sem: pltpu.async_copy(
          tbl_h.at[q].at[idx], dst, sem)
      _edge_loop(_K2, _G2, _B2, gather, tbl_h.at[q].at[pl.ds(0, 128)],
                 srcR_h, dstR_h, acc, isrc, idst, rows,
                 (gsem0, gsem1), (ssem0, ssem1), s * (_ER // 16))
      plsc.subcore_barrier()
      pltpu.sync_copy(acc.at[pl.ds(s * _ZR, _ZR)],
                      out_h.at[pl.ds(s * _ZR, _ZR), pl.ds(q * 32, 32)])
      plsc.subcore_barrier()

    @pl.when(c == 0)
    def _():
      split(0)
      split(1)
      run_pass(0)
      run_pass(1)

    @pl.when(c == 1)
    def _():
      split(2)
      split(3)
      run_pass(2)
      run_pass(3)

  return k(h1, srcR, dstR, z32)[0]


def _ln(h, g, b):
  mu = jnp.mean(h, axis=-1, keepdims=True)
  var = jnp.mean((h - mu) ** 2, axis=-1, keepdims=True)
  return (h - mu) * lax.rsqrt(var + 1e-5) * g + b


_HI = lax.Precision.HIGHEST


def _tc_layer1(msum1, xp, wl, wr, b1, g1, be1):
  def body(ms_ref, xp_ref, wl_ref, wr_ref, b_ref, g_ref, be_ref, o_ref):
    sm = ms_ref[:, 0:8] + ms_ref[:, 8:16]
    r = 1.0 / jnp.maximum(sm[:, 6:7], 1.0)
    mean8 = sm * r
    lin = jnp.dot(mean8, wl_ref[...], preferred_element_type=jnp.float32,
                  precision=_HI)
    lin = lin + jnp.dot(xp_ref[...], wr_ref[...],
                        preferred_element_type=jnp.float32, precision=_HI)
    lin = lin + b_ref[...]
    o_ref[...] = jnp.maximum(_ln(lin, g_ref[...], be_ref[...]), 0.0)

  full = lambda shape: pl.BlockSpec(shape, lambda i: (0,) * len(shape))
  return pl.pallas_call(
      body,
      grid=(_NB,),
      in_specs=[
          pl.BlockSpec((_BN, 16), lambda i: (i, 0)),
          pl.BlockSpec((_BN, 8), lambda i: (i, 0)),
          full((8, 128)), full((8, 128)),
          full((1, 128)), full((1, 128)), full((1, 128)),
      ],
      out_specs=pl.BlockSpec((_BN, 128), lambda i: (i, 0)),
      out_shape=jax.ShapeDtypeStruct((_NTC, 128), jnp.float32),
  )(msum1, xp, wl, wr, b1, g1, be1)


def _tc_ffm(crd, bt, bf):
  def body(crdr, btr, bfr, out, acc):
    i = pl.program_id(0)

    @pl.when(i == 0)
    def _():
      acc[...] = jnp.zeros_like(acc)

    crd_blk = crdr[...]
    parts = []
    for sfac in _SCALES:
      ps = jnp.dot(crd_blk * sfac, bfr[...], preferred_element_type=jnp.float32)
      parts.append(jnp.sin(ps))
      parts.append(jnp.cos(ps))
    mk = (lax.broadcasted_iota(jnp.int32, (_BN, 128), 1) == 0)
    parts.append(mk.astype(jnp.float32))
    hcat = jnp.concatenate(parts, axis=1)

    bvec = btr[0, 0, :]
    oneh = (lax.broadcasted_iota(jnp.int32, (_G, _BN), 0)
            == bvec[None, :]).astype(jnp.float32)
    acc[...] += jnp.dot(oneh, hcat, preferred_element_type=jnp.float32,
                        precision=_HI)

    @pl.when(i == _NB - 1)
    def _():
      out[...] = acc[...]

  full = lambda shape: pl.BlockSpec(shape, lambda i: (0,) * len(shape))
  return pl.pallas_call(
      body,
      grid=(_NB,),
      in_specs=[
          pl.BlockSpec((_BN, 4), lambda i: (i, 0)),
          pl.BlockSpec((1, 1, _BN), lambda i: (i, 0, 0)),
          full((4, 128)),
      ],
      out_specs=pl.BlockSpec((_G, 1152), lambda i: (0, 0)),
      out_shape=jax.ShapeDtypeStruct((_G, 1152), jnp.float32),
      scratch_shapes=[pltpu.VMEM((_G, 1152), jnp.float32)],
  )(crd, bt, bf)


def _tc_final(msum2, msum1, h1, bt, ffmp, w2l, w2r, b2, g2, be2,
              fcw, fcb2, fg, fb):
  def body(ms2, ms1, h1r, btr, ffr, w2lr, w2rr, b2r, g2r, be2r, fcwr, fcbr,
           fgr, fbr, out, acc):
    i = pl.program_id(0)

    @pl.when(i == 0)
    def _():
      acc[...] = jnp.zeros_like(acc)

    sm1 = ms1[:, 0:8] + ms1[:, 8:16]
    r = 1.0 / jnp.maximum(sm1[:, 6:7], 1.0)
    mean2 = ms2[...] * r
    lin = jnp.dot(mean2, w2lr[...], preferred_element_type=jnp.float32,
                  precision=_HI)
    lin = lin + jnp.dot(h1r[...], w2rr[...], preferred_element_type=jnp.float32,
                        precision=_HI)
    lin = lin + b2r[...]
    h2 = jnp.maximum(_ln(lin, g2r[...], be2r[...]), 0.0)

    bvec = btr[0, 0, :]
    oneh = (lax.broadcasted_iota(jnp.int32, (_G, _BN), 0)
            == bvec[None, :]).astype(jnp.float32)
    acc[...] += jnp.dot(oneh, h2, preferred_element_type=jnp.float32,
                        precision=_HI)

    @pl.when(i == _NB - 1)
    def _():
      fp = ffr[...]
      rg = 1.0 / jnp.maximum(fp[:, 1024:1025], 1.0)
      g = jnp.concatenate([acc[...], fp[:, 0:1024]], axis=1) * rg
      o = jnp.dot(g, fcwr[...], preferred_element_type=jnp.float32,
                  precision=_HI)
      o = o + fcbr[...]
      out[...] = _ln(o, fgr[...], fbr[...])

  full = lambda shape: pl.BlockSpec(shape, lambda i: (0,) * len(shape))
  return pl.pallas_call(
      body,
      grid=(_NB,),
      in_specs=[
          pl.BlockSpec((_BN, 128), lambda i: (i, 0)),
          pl.BlockSpec((_BN, 16), lambda i: (i, 0)),
          pl.BlockSpec((_BN, 128), lambda i: (i, 0)),
          pl.BlockSpec((1, 1, _BN), lambda i: (i, 0, 0)),
          full((_G, 1152)),
          full((128, 128)), full((128, 128)),
          full((1, 128)), full((1, 128)), full((1, 128)),
          full((1152, 256)), full((1, 256)), full((1, 256)), full((1, 256)),
      ],
      out_specs=pl.BlockSpec((_G, 256), lambda i: (0, 0)),
      out_shape=jax.ShapeDtypeStruct((_G, 256), jnp.float32),
      scratch_shapes=[pltpu.VMEM((_G, 128), jnp.float32)],
  )(msum2, msum1, h1, bt, ffmp, w2l, w2r, b2, g2, be2, fcw, fcb2, fg, fb)


def kernel(x, edge_index, node_coords, batch, W1l, b1l, W1r, W2l, b2l, W2r,
           ln1g, ln1b, ln2g, ln2b, Bffm, fcW, fcb, flng, flnb):
  f32 = jnp.float32
  src = edge_index[0]
  dst = edge_index[1]
  npad = _EPAD - _E
  pidx = lax.iota(jnp.int32, npad) % 16
  ztail = jnp.zeros(((_ERX - _ER) * 128,), jnp.int32)
  srcR = jnp.concatenate([src, pidx, ztail]).reshape(_ERX, 128)
  dstR = jnp.concatenate([dst, _N + pidx, ztail]).reshape(_ERX, 128)

  xp = jnp.concatenate(
      [x, jnp.ones((_N, 1), f32), jnp.zeros((_N, 1), f32)], axis=1)
  xp = jnp.pad(xp, ((0, _NTC - _N), (0, 0)))
  z8 = jnp.zeros((_ZR, 8), f32)
  z32 = jnp.zeros((_ZR, 32), f32)

  msum1 = _sc_layer1(xp, srcR, dstR, z8)

  wl1 = jnp.pad(W1l.T, ((0, 2), (0, 0)))
  wr1 = jnp.pad(W1r.T, ((0, 2), (0, 0)))
  h1 = _tc_layer1(msum1, xp, wl1, wr1, b1l.reshape(1, 128),
                  ln1g.reshape(1, 128), ln1b.reshape(1, 128))

  msum2 = _sc_layer2(h1, srcR, dstR, z32)

  crd = jnp.pad(node_coords, ((0, _NTC - _N), (0, 1)))
  bt = jnp.pad(batch, (0, _NTC - _N), constant_values=_G).reshape(_NB, 1, _BN)
  bf = jnp.pad(Bffm, ((0, 1), (0, 0)))
  ffmp = _tc_ffm(crd, bt, bf)

  out = _tc_final(msum2, msum1, h1, bt, ffmp, W2l.T, W2r.T,
                  b2l.reshape(1, 128), ln2g.reshape(1, 128),
                  ln2b.reshape(1, 128), fcW.T, fcb.reshape(1, 256),
                  flng.reshape(1, 256), flnb.reshape(1, 256))
  return out


# speedup vs baseline: 8.4924x; 1.0389x over previous
import functools

import jax
import jax.numpy as jnp
from jax import lax
from jax.experimental import pallas as pl
from jax.experimental.pallas import tpu as pltpu
from jax.experimental.pallas import tpu_sc as plsc

_N = 50000
_E = 800000
_G = 64
_BN = 256
_NB = 196
_NTC = _NB * _BN
_NPAD = 51200
_ZR = _NPAD // 16
_ER = 6400
_ERX = _ER + 64
_EPAD = _ER * 128
_K1 = 4
_G1 = 5
_B1 = 10
_K2 = 2
_G2 = 10
_B2 = 20
_SCALES = (1.0, 5.0, 10.0, 30.0)


def _sc_mesh():
  return plsc.VectorSubcoreMesh(
      core_axis_name="c", subcore_axis_name="s", num_cores=2, num_subcores=16)


def _edge_loop(k, gpb, n_blocks, gather_fn, drain_src, srcR, dstR, acc,
               isrc, idst, rows, gsems, ssems, base_row):
  rpb = gpb * k

  def stage(blk, ib):
    row0 = base_row + blk * rpb
    pltpu.sync_copy(srcR.at[pl.ds(row0, rpb)], isrc.at[ib])
    pltpu.sync_copy(dstR.at[pl.ds(row0, rpb)], idst.at[ib])

  pend = [None, None]

  def fire(g, ib, t):
    for j in range(k):
      gather_fn(isrc.at[ib].at[t * k + j], rows.at[g].at[j], gsems[g])
    pend[g] = (ib, t)

  def finish(g):
    ib, t = pend[g]
    for j in range(k):
      pltpu.make_async_copy(drain_src, rows.at[g].at[j], gsems[g]).wait()
    sd = [pltpu.async_copy(rows.at[g].at[j], acc.at[idst.at[ib].at[t * k + j]],
                           ssems[g], add=True) for j in range(k)]
    for d in sd:
      d.wait()

  def run_block(ib, next_blk):
    for t in range(2, gpb):
      g = t % 2
      finish(g)
      fire(g, ib, t)
    if next_blk is not None:
      stage(next_blk, 1 - ib)
      for t in range(2):
        finish(t)
        fire(t, 1 - ib, t)

  stage(0, 0)
  fire(0, 0, 0)
  fire(1, 0, 1)

  def pair(i, carry):
    run_block(0, 2 * i + 1)
    run_block(1, 2 * i + 2)
    return carry

  lax.fori_loop(0, n_blocks // 2 - 1, pair, 0)
  run_block(0, n_blocks - 1)
  run_block(1, None)
  finish(0)
  finish(1)


def _sc_layer1(xp, srcR, dstR, z8):
  @functools.partial(
      pl.kernel,
      out_type=jax.ShapeDtypeStruct((_NPAD, 16), jnp.float32),
      mesh=_sc_mesh(),
      compiler_params=pltpu.CompilerParams(use_tc_tiling_on_sc=False),
      scratch_types=[
          pltpu.VMEM_SHARED((_NPAD, 8), jnp.float32),
          pltpu.VMEM((2, _G1 * _K1, 128), jnp.int32),
          pltpu.VMEM((2, _G1 * _K1, 128), jnp.int32),
          pltpu.VMEM((2, _K1, 128, 8), jnp.float32),
          pltpu.SemaphoreType.DMA,
          pltpu.SemaphoreType.DMA,
          pltpu.SemaphoreType.DMA,
          pltpu.SemaphoreType.DMA,
      ],
  )
  def k(xp_h, srcR_h, dstR_h, z8_h, out_h, acc, isrc, idst, rows,
        gsem0, gsem1, ssem0, ssem1):
    c = lax.axis_index("c")
    s = lax.axis_index("s")
    pltpu.sync_copy(z8_h, acc.at[pl.ds(s * _ZR, _ZR)])
    plsc.subcore_barrier()
    rows_per_tile = (_ER // 2) // 16
    base = c * (_ER // 2) + s * rows_per_tile
    gather = lambda idx, dst, sem: pltpu.async_copy(xp_h.at[idx], dst, sem)
    _edge_loop(_K1, _G1, _B1, gather, xp_h.at[pl.ds(0, 128)], srcR_h, dstR_h,
               acc, isrc, idst, rows, (gsem0, gsem1), (ssem0, ssem1), base)
    plsc.subcore_barrier()

    @pl.when(c == 0)
    def _():
      pltpu.sync_copy(acc.at[pl.ds(s * _ZR, _ZR)],
                      out_h.at[pl.ds(s * _ZR, _ZR), pl.ds(0, 8)])

    @pl.when(c == 1)
    def _():
      pltpu.sync_copy(acc.at[pl.ds(s * _ZR, _ZR)],
                      out_h.at[pl.ds(s * _ZR, _ZR), pl.ds(8, 8)])

  return k(xp, srcR, dstR, z8)


_TROWS = _NTC // 16
_TSTEP = 112


def _sc_layer2(h1, srcR, dstR, z32):
  @functools.partial(
      pl.kernel,
      out_type=(jax.ShapeDtypeStruct((_NPAD, 128), jnp.float32),
                jax.ShapeDtypeStruct((4, _NTC, 32), jnp.float32)),
      mesh=_sc_mesh(),
      compiler_params=pltpu.CompilerParams(use_tc_tiling_on_sc=False),
      scratch_types=[
          pltpu.VMEM_SHARED((_NPAD, 32), jnp.float32),
          pltpu.VMEM((2, _G2 * _K2, 128), jnp.int32),
          pltpu.VMEM((2, _G2 * _K2, 128), jnp.int32),
          pltpu.VMEM((2, _K2, 128, 32), jnp.float32),
          pltpu.SemaphoreType.DMA,
          pltpu.SemaphoreType.DMA,
          pltpu.SemaphoreType.DMA,
          pltpu.SemaphoreType.DMA,
      ],
  )
  def k(h1_h, srcR_h, dstR_h, z32_h, out_h, tbl_h,
        acc, isrc, idst, rows, gsem0, gsem1, ssem0, ssem1):
    c = lax.axis_index("c")
    s = lax.axis_index("s")

    def split(qj):
      stage = rows.at[0, 0, pl.ds(0, _TSTEP)]
      def it(i, carry):
        r0 = s * _TROWS + i * _TSTEP
        pltpu.sync_copy(h1_h.at[pl.ds(r0, _TSTEP), pl.ds(qj * 32, 32)], stage)
        pltpu.sync_copy(stage, tbl_h.at[qj].at[pl.ds(r0, _TSTEP)])
        return carry
      lax.fori_loop(0, _TROWS // _TSTEP, it, 0)

    def run_pass(q):
      pltpu.sync_copy(z32_h, acc.at[pl.ds(s * _ZR, _ZR)])
      plsc.subcore_barrier()
      gather = lambda idx, dst, sem: pltpu.async_copy(
          tbl_h.at[q].at[idx], dst, sem)
      _edge_loop(_K2, _G2, _B2, gather, tbl_h.at[q].at[pl.ds(0, 128)],
                 srcR_h, dstR_h, acc, isrc, idst, rows,
                 (gsem0, gsem1), (ssem0, ssem1), s * (_ER // 16))
      plsc.subcore_barrier()
      pltpu.sync_copy(acc.at[pl.ds(s * _ZR, _ZR)],
                      out_h.at[pl.ds(s * _ZR, _ZR), pl.ds(q * 32, 32)])
      plsc.subcore_barrier()

    @pl.when(c == 0)
    def _():
      split(0)
      split(1)
      run_pass(0)
      run_pass(1)

    @pl.when(c == 1)
    def _():
      split(2)
      split(3)
      run_pass(2)
      run_pass(3)

  return k(h1, srcR, dstR, z32)[0]


def _ln(h, g, b):
  mu = jnp.mean(h, axis=-1, keepdims=True)
  var = jnp.mean((h - mu) ** 2, axis=-1, keepdims=True)
  return (h - mu) * lax.rsqrt(var + 1e-5) * g + b


_HI = lax.Precision.HIGHEST


def _tc_layer1(msum1, xp, wl, wr, b1, g1, be1):
  def body(ms_ref, xp_ref, wl_ref, wr_ref, b_ref, g_ref, be_ref, o_ref):
    sm = ms_ref[:, 0:8] + ms_ref[:, 8:16]
    r = 1.0 / jnp.maximum(sm[:, 6:7], 1.0)
    mean8 = sm * r
    lin = jnp.dot(mean8, wl_ref[...], preferred_element_type=jnp.float32,
                  precision=_HI)
    lin = lin + jnp.dot(xp_ref[...], wr_ref[...],
                        preferred_element_type=jnp.float32, precision=_HI)
    lin = lin + b_ref[...]
    o_ref[...] = jnp.maximum(_ln(lin, g_ref[...], be_ref[...]), 0.0)

  full = lambda shape: pl.BlockSpec(shape, lambda i: (0,) * len(shape))
  return pl.pallas_call(
      body,
      grid=(_NB,),
      in_specs=[
          pl.BlockSpec((_BN, 16), lambda i: (i, 0)),
          pl.BlockSpec((_BN, 8), lambda i: (i, 0)),
          full((8, 128)), full((8, 128)),
          full((1, 128)), full((1, 128)), full((1, 128)),
      ],
      out_specs=pl.BlockSpec((_BN, 128), lambda i: (i, 0)),
      out_shape=jax.ShapeDtypeStruct((_NTC, 128), jnp.float32),
  )(msum1, xp, wl, wr, b1, g1, be1)


def _tc_ffm(crd, bt, bf):
  def body(crdr, btr, bfr, out, acc):
    i = pl.program_id(0)

    @pl.when(i == 0)
    def _():
      acc[...] = jnp.zeros_like(acc)

    crd_blk = crdr[...]
    parts = []
    for sfac in _SCALES:
      ps = jnp.dot(crd_blk * sfac, bfr[...], preferred_element_type=jnp.float32)
      parts.append(jnp.sin(ps))
      parts.append(jnp.cos(ps))
    mk = (lax.broadcasted_iota(jnp.int32, (_BN, 128), 1) == 0)
    parts.append(mk.astype(jnp.float32))
    hcat = jnp.concatenate(parts, axis=1)

    bvec = btr[0, 0, :]
    oneh = (lax.broadcasted_iota(jnp.int32, (_G, _BN), 0)
            == bvec[None, :]).astype(jnp.float32)
    acc[...] += jnp.dot(oneh, hcat, preferred_element_type=jnp.float32)

    @pl.when(i == _NB - 1)
    def _():
      out[...] = acc[...]

  full = lambda shape: pl.BlockSpec(shape, lambda i: (0,) * len(shape))
  return pl.pallas_call(
      body,
      grid=(_NB,),
      in_specs=[
          pl.BlockSpec((_BN, 4), lambda i: (i, 0)),
          pl.BlockSpec((1, 1, _BN), lambda i: (i, 0, 0)),
          full((4, 128)),
      ],
      out_specs=pl.BlockSpec((_G, 1152), lambda i: (0, 0)),
      out_shape=jax.ShapeDtypeStruct((_G, 1152), jnp.float32),
      scratch_shapes=[pltpu.VMEM((_G, 1152), jnp.float32)],
  )(crd, bt, bf)


def _tc_final(msum2, msum1, h1, bt, ffmp, w2l, w2r, b2, g2, be2,
              fcw, fcb2, fg, fb):
  def body(ms2, ms1, h1r, btr, ffr, w2lr, w2rr, b2r, g2r, be2r, fcwr, fcbr,
           fgr, fbr, out, acc):
    i = pl.program_id(0)

    @pl.when(i == 0)
    def _():
      acc[...] = jnp.zeros_like(acc)

    sm1 = ms1[:, 0:8] + ms1[:, 8:16]
    r = 1.0 / jnp.maximum(sm1[:, 6:7], 1.0)
    mean2 = ms2[...] * r
    lin = jnp.dot(mean2, w2lr[...], preferred_element_type=jnp.float32)
    lin = lin + jnp.dot(h1r[...], w2rr[...], preferred_element_type=jnp.float32)
    lin = lin + b2r[...]
    h2 = jnp.maximum(_ln(lin, g2r[...], be2r[...]), 0.0)

    bvec = btr[0, 0, :]
    oneh = (lax.broadcasted_iota(jnp.int32, (_G, _BN), 0)
            == bvec[None, :]).astype(jnp.float32)
    acc[...] += jnp.dot(oneh, h2, preferred_element_type=jnp.float32)

    @pl.when(i == _NB - 1)
    def _():
      fp = ffr[...]
      rg = 1.0 / jnp.maximum(fp[:, 1024:1025], 1.0)
      g = jnp.concatenate([acc[...], fp[:, 0:1024]], axis=1) * rg
      o = jnp.dot(g, fcwr[...], preferred_element_type=jnp.float32,
                  precision=_HI)
      o = o + fcbr[...]
      out[...] = _ln(o, fgr[...], fbr[...])

  full = lambda shape: pl.BlockSpec(shape, lambda i: (0,) * len(shape))
  return pl.pallas_call(
      body,
      grid=(_NB,),
      in_specs=[
          pl.BlockSpec((_BN, 128), lambda i: (i, 0)),
          pl.BlockSpec((_BN, 16), lambda i: (i, 0)),
          pl.BlockSpec((_BN, 128), lambda i: (i, 0)),
          pl.BlockSpec((1, 1, _BN), lambda i: (i, 0, 0)),
          full((_G, 1152)),
          full((128, 128)), full((128, 128)),
          full((1, 128)), full((1, 128)), full((1, 128)),
          full((1152, 256)), full((1, 256)), full((1, 256)), full((1, 256)),
      ],
      out_specs=pl.BlockSpec((_G, 256), lambda i: (0, 0)),
      out_shape=jax.ShapeDtypeStruct((_G, 256), jnp.float32),
      scratch_shapes=[pltpu.VMEM((_G, 128), jnp.float32)],
  )(msum2, msum1, h1, bt, ffmp, w2l, w2r, b2, g2, be2, fcw, fcb2, fg, fb)


def kernel(x, edge_index, node_coords, batch, W1l, b1l, W1r, W2l, b2l, W2r,
           ln1g, ln1b, ln2g, ln2b, Bffm, fcW, fcb, flng, flnb):
  f32 = jnp.float32
  src = edge_index[0]
  dst = edge_index[1]
  npad = _EPAD - _E
  pidx = lax.iota(jnp.int32, npad) % 16
  ztail = jnp.zeros(((_ERX - _ER) * 128,), jnp.int32)
  srcR = jnp.concatenate([src, pidx, ztail]).reshape(_ERX, 128)
  dstR = jnp.concatenate([dst, _N + pidx, ztail]).reshape(_ERX, 128)

  xp = jnp.concatenate(
      [x, jnp.ones((_N, 1), f32), jnp.zeros((_N, 1), f32)], axis=1)
  xp = jnp.pad(xp, ((0, _NTC - _N), (0, 0)))
  z8 = jnp.zeros((_ZR, 8), f32)
  z32 = jnp.zeros((_ZR, 32), f32)

  msum1 = _sc_layer1(xp, srcR, dstR, z8)

  wl1 = jnp.pad(W1l.T, ((0, 2), (0, 0)))
  wr1 = jnp.pad(W1r.T, ((0, 2), (0, 0)))
  h1 = _tc_layer1(msum1, xp, wl1, wr1, b1l.reshape(1, 128),
                  ln1g.reshape(1, 128), ln1b.reshape(1, 128))

  msum2 = _sc_layer2(h1, srcR, dstR, z32)

  crd = jnp.pad(node_coords, ((0, _NTC - _N), (0, 1)))
  bt = jnp.pad(batch, (0, _NTC - _N), constant_values=_G).reshape(_NB, 1, _BN)
  bf = jnp.pad(Bffm, ((0, 1), (0, 0)))
  ffmp = _tc_ffm(crd, bt, bf)

  out = _tc_final(msum2, msum1, h1, bt, ffmp, W2l.T, W2r.T,
                  b2l.reshape(1, 128), ln2g.reshape(1, 128),
                  ln2b.reshape(1, 128), fcW.T, fcb.reshape(1, 256),
                  flng.reshape(1, 256), flnb.reshape(1, 256))
  return out
